# baseline (speedup 1.0000x reference)
import functools

import jax
import jax.numpy as jnp
from jax import lax
from jax.experimental import pallas as pl
from jax.experimental.pallas import tpu as pltpu
from jax.experimental.pallas import tpu_sc as plsc

N = 50000
E = 1600000
R = 16
NB = 30
H = 16
C = 8

NC = 2
NS = 16
NW = NC * NS
RH = R // NC
EPW = E // NW
EPT = E // NS
CH = 400
CHC = 2000
NROW = 3128
NROW_L = N - (NS - 1) * NROW

_MESH = plsc.VectorSubcoreMesh(core_axis_name="c", subcore_axis_name="s",
                               num_cores=NC, num_subcores=NS)
_SC_PARAMS = pltpu.CompilerParams(needs_layout_passes=False,
                                  use_tc_tiling_on_sc=False)


def _w1_body(b3_ref, m2_ref, out_ref):
    out_ref[...] = lax.dot_general(
        b3_ref[...], m2_ref[...],
        ((((0,), (1,)), ((), ()))),
        preferred_element_type=jnp.float32)


def _w1_call(b3, m2):
    blk = 1280
    grid = (N + blk - 1) // blk
    return pl.pallas_call(
        _w1_body,
        grid=(grid,),
        in_specs=[pl.BlockSpec((NB * H, blk), lambda i: (0, i)),
                  pl.BlockSpec((R * H, NB * H), lambda i: (0, 0))],
        out_specs=pl.BlockSpec((blk, R * H), lambda i: (i, 0)),
        out_shape=jax.ShapeDtypeStruct((N, R * H), jnp.float32),
    )(b3, m2)


@functools.partial(
    pl.kernel,
    out_type=jax.ShapeDtypeStruct((NC * N * RH,), jnp.float32),
    mesh=_MESH,
    compiler_params=_SC_PARAMS,
    scratch_types=[
        pltpu.VMEM((CHC,), jnp.int32),
        pltpu.VMEM((CHC,), jnp.int32),
        pltpu.VMEM((CHC,), jnp.int32),
        pltpu.VMEM((CHC,), jnp.float32),
        pltpu.SemaphoreType.DMA,
        pltpu.SemaphoreType.DMA,
        pltpu.VMEM((25600,), jnp.float32),
        pltpu.VMEM_SHARED((N * RH,), jnp.float32),
    ],
)
def _count_kernel(dst_hbm, typ_hbm, out_hbm,
                  dst_v, typ_v, idx_v, ones_v, semA, semB, cbuf, cnt_sh):
    cid = lax.axis_index("c")
    sid = lax.axis_index("s")
    base = sid * EPT
    zsl = (N * RH) // NS

    def zero_body(j, carry):
        cbuf[pl.ds(j * 16, 16)] = jnp.zeros((16,), jnp.float32)
        return carry
    lax.fori_loop(0, 25600 // 16, zero_body, 0)

    def fill_body(j, carry):
        ones_v[pl.ds(j * 16, 16)] = jnp.full((16,), 1.0, jnp.float32)
        return carry
    lax.fori_loop(0, CHC // 16, fill_body, 0)
    pltpu.sync_copy(cbuf.at[pl.ds(0, zsl)], cnt_sh.at[pl.ds(sid * zsl, zsl)])
    plsc.subcore_barrier()

    tlo = cid * RH

    def chunk_body(k, carry):
        off = base + k * CHC
        cpa = pltpu.async_copy(dst_hbm.at[pl.ds(off, CHC)], dst_v, semA)
        cpb = pltpu.async_copy(typ_hbm.at[pl.ds(off, CHC)], typ_v, semB)
        cpa.wait()
        cpb.wait()

        def vec_body(j, c2):
            d = dst_v[pl.ds(j * 16, 16)]
            t = typ_v[pl.ds(j * 16, 16)] - tlo
            ok = (t >= 0) & (t < RH)
            idx_v[pl.ds(j * 16, 16)] = jnp.where(ok, d * RH + t, -1)
            return c2
        lax.fori_loop(0, CHC // 16, vec_body, 0)
        pltpu.sync_copy(
            ones_v,
            cnt_sh.at[plsc.Indices(idx_v, ignored_value=-1)],
            add=True)
        return carry
    lax.fori_loop(0, EPT // CHC, chunk_body, 0)
    plsc.subcore_barrier()
    pltpu.sync_copy(cnt_sh.at[pl.ds(sid * zsl, zsl)], cbuf.at[pl.ds(0, zsl)])
    pltpu.sync_copy(cbuf.at[pl.ds(0, zsl)],
                    out_hbm.at[pl.ds(cid * N * RH + sid * zsl, zsl)])


def _invn_body(cnt_ref, out_ref):
    lo = 1.0 / jnp.maximum(cnt_ref[0], 1.0)
    hi = 1.0 / jnp.maximum(cnt_ref[1], 1.0)
    out_ref[...] = jnp.concatenate([lo, hi], axis=1)


def _invn_call(cnt3):
    blk = 5000
    return pl.pallas_call(
        _invn_body,
        grid=(N // blk,),
        in_specs=[pl.BlockSpec((NC, blk, RH), lambda i: (0, i, 0))],
        out_specs=pl.BlockSpec((blk, R), lambda i: (i, 0)),
        out_shape=jax.ShapeDtypeStruct((N, R), jnp.float32),
    )(cnt3)


@functools.partial(
    pl.kernel,
    out_type=(jax.ShapeDtypeStruct((NC, N, H), jnp.float32),
              jax.ShapeDtypeStruct((E,), jnp.float32)),
    mesh=_MESH,
    compiler_params=_SC_PARAMS,
    scratch_types=[
        pltpu.VMEM((CH,), jnp.int32),
        pltpu.VMEM((CH,), jnp.int32),
        pltpu.VMEM((CH,), jnp.int32),
        pltpu.VMEM((CH,), jnp.int32),
        pltpu.VMEM((CH, H), jnp.float32),
        pltpu.VMEM((CH, R), jnp.float32),
        pltpu.VMEM((CH,), jnp.float32),
        pltpu.VMEM((NROW, H), jnp.float32),
        pltpu.VMEM_SHARED((N, H), jnp.float32),
        pltpu.SemaphoreType.DMA,
        pltpu.SemaphoreType.DMA,
        pltpu.SemaphoreType.DMA,
        pltpu.SemaphoreType.DMA,
        pltpu.SemaphoreType.DMA,
    ],
)
def _agg1_kernel(src_hbm, dst_hbm, typ_hbm, w1_hbm, invn_hbm,
                 agg_out, inve_out,
                 src_v, dst_v, typ_v, idx_v, rows_v, invr_v, inve_v,
                 obuf, agg_sh, sem1, sem2, semA, semB, semC):
    cid = lax.axis_index("c")
    sid = lax.axis_index("s")
    wid = sid * NC + cid
    base = wid * EPW
    st = sid * NROW

    def zrow_body(r, carry):
        obuf[r, :] = jnp.zeros((16,), jnp.float32)
        return carry
    lax.fori_loop(0, NROW, zrow_body, 0)

    @pl.when(sid < NS - 1)
    def _():
        pltpu.sync_copy(obuf, agg_sh.at[pl.ds(st, NROW)])

    @pl.when(sid == NS - 1)
    def _():
        pltpu.sync_copy(obuf.at[pl.ds(0, NROW_L)],
                        agg_sh.at[pl.ds(st, NROW_L)])
    plsc.subcore_barrier()

    def chunk_body(k, carry):
        off = base + k * CH
        cpa = pltpu.async_copy(src_hbm.at[pl.ds(off, CH)], src_v, semA)
        cpb = pltpu.async_copy(dst_hbm.at[pl.ds(off, CH)], dst_v, semB)
        cpc = pltpu.async_copy(typ_hbm.at[pl.ds(off, CH)], typ_v, semC)
        cpb.wait()
        cp1 = pltpu.async_copy(invn_hbm.at[dst_v], invr_v, sem1)
        cpa.wait()
        cpc.wait()

        def idx_body(j, c2):
            s = src_v[pl.ds(j * 16, 16)]
            t = typ_v[pl.ds(j * 16, 16)]
            idx_v[pl.ds(j * 16, 16)] = s * R + t
            return c2
        lax.fori_loop(0, CH // 16, idx_body, 0)
        cp2 = pltpu.async_copy(w1_hbm.at[idx_v], rows_v, sem2)
        cp1.wait()

        def ext_body(j, c2):
            t = typ_v[pl.ds(j * 16, 16)]
            r = jnp.arange(16, dtype=jnp.int32) + j * 16
            inve_v[pl.ds(j * 16, 16)] = plsc.load_gather(invr_v, [r, t])
            return c2
        lax.fori_loop(0, CH // 16, ext_body, 0)
        cp2.wait()

        def scale_body(g, c2):
            iv = inve_v[pl.ds(g * 16, 16)]
            for l in range(16):
                e = g * 16 + l
                rows_v[e, :] = rows_v[e, :] * iv[l]
            return c2
        lax.fori_loop(0, CH // 16, scale_body, 0)
        pltpu.sync_copy(rows_v, agg_sh.at[dst_v], add=True)
        pltpu.sync_copy(inve_v, inve_out.at[pl.ds(off, CH)])
        return carry
    lax.fori_loop(0, EPW // CH, chunk_body, 0)
    plsc.subcore_barrier()

    @pl.when(sid < NS - 1)
    def _():
        pltpu.sync_copy(agg_sh.at[pl.ds(st, NROW)], obuf)
        pltpu.sync_copy(obuf, agg_out.at[cid, pl.ds(st, NROW)])

    @pl.when(sid == NS - 1)
    def _():
        pltpu.sync_copy(agg_sh.at[pl.ds(st, NROW_L)],
                        obuf.at[pl.ds(0, NROW_L)])
        pltpu.sync_copy(obuf.at[pl.ds(0, NROW_L)],
                        agg_out.at[cid, pl.ds(st, NROW_L)])


def _mid_body(aggp_ref, root_ref, bias_ref, basis2t_ref, m_ref,
              h_ref, xw_ref):
    h = jnp.maximum(
        aggp_ref[0] + aggp_ref[1] + root_ref[...] + bias_ref[...], 0.0)
    h_ref[...] = h
    hb = jnp.dot(h, basis2t_ref[...], preferred_element_type=jnp.float32)
    xw_ref[...] = jnp.dot(hb, m_ref[...], preferred_element_type=jnp.float32)


def _mid_call(agg1_p, root1, bias1_2d, basis2_t, m_mat):
    blk = 2000
    return pl.pallas_call(
        _mid_body,
        grid=(N // blk,),
        in_specs=[pl.BlockSpec((NC, blk, H), lambda i: (0, i, 0)),
                  pl.BlockSpec((blk, H), lambda i: (i, 0)),
                  pl.BlockSpec((1, H), lambda i: (0, 0)),
                  pl.BlockSpec((H, NB * C), lambda i: (0, 0)),
                  pl.BlockSpec((NB * C, R * C), lambda i: (0, 0))],
        out_specs=(pl.BlockSpec((blk, H), lambda i: (i, 0)),
                   pl.BlockSpec((blk, R * C), lambda i: (i, 0))),
        out_shape=(jax.ShapeDtypeStruct((N, H), jnp.float32),
                   jax.ShapeDtypeStruct((N, R * C), jnp.float32)),
    )(agg1_p, root1, bias1_2d, basis2_t, m_mat)


_NCH2 = EPW // CH


@functools.partial(
    pl.kernel,
    out_type=jax.ShapeDtypeStruct((NC, N, C), jnp.float32),
    mesh=_MESH,
    compiler_params=_SC_PARAMS,
    scratch_types=(
        [pltpu.VMEM((CH,), jnp.int32)] * 8 +
        [pltpu.VMEM((CH,), jnp.float32)] * 2 +
        [pltpu.VMEM((CH, C), jnp.float32)] * 2 +
        [pltpu.VMEM((NROW, C), jnp.float32),
         pltpu.VMEM_SHARED((N, C), jnp.float32)] +
        [pltpu.SemaphoreType.DMA] * 4
    ),
)
def _agg2_kernel(src_hbm, dst_hbm, typ_hbm, xw_hbm, inve_hbm,
                 agg_out,
                 src0, src1, dst0, dst1, typ0, typ1, idx0, idx1,
                 inv0, inv1, rows0, rows1, obuf, agg_sh,
                 semL0, semL1, semG0, semG1):
    cid = lax.axis_index("c")
    sid = lax.axis_index("s")
    wid = sid * NC + cid
    base = wid * EPW
    st = sid * NROW
    lane = jnp.arange(16, dtype=jnp.int32)
    colq = lane & 7
    rowq = lane >> 3
    srcs = [src0, src1]
    dsts = [dst0, dst1]
    typs = [typ0, typ1]
    idxs = [idx0, idx1]
    invs = [inv0, inv1]
    rows = [rows0, rows1]
    semL = [semL0, semL1]
    semG = [semG0, semG1]

    def zrow_body(g, carry):
        rid = g * 2 + rowq
        plsc.store_scatter(obuf, [rid, colq], jnp.zeros((16,), jnp.float32))
        return carry
    lax.fori_loop(0, NROW // 2, zrow_body, 0)

    @pl.when(sid < NS - 1)
    def _():
        pltpu.sync_copy(obuf, agg_sh.at[pl.ds(st, NROW)])

    @pl.when(sid == NS - 1)
    def _():
        pltpu.sync_copy(obuf.at[pl.ds(0, NROW_L)],
                        agg_sh.at[pl.ds(st, NROW_L)])
    plsc.subcore_barrier()

    def issue_linear(b, k):
        off = base + k * CH
        pltpu.async_copy(src_hbm.at[pl.ds(off, CH)], srcs[b], semL[b])
        pltpu.async_copy(dst_hbm.at[pl.ds(off, CH)], dsts[b], semL[b])
        pltpu.async_copy(typ_hbm.at[pl.ds(off, CH)], typs[b], semL[b])
        pltpu.async_copy(inve_hbm.at[pl.ds(off, CH)], invs[b], semL[b])

    def prep_idx(b, k):
        off = base + k * CH
        pltpu.make_async_copy(src_hbm.at[pl.ds(off, CH)], srcs[b],
                              semL[b]).wait()
        pltpu.make_async_copy(dst_hbm.at[pl.ds(off, CH)], dsts[b],
                              semL[b]).wait()
        pltpu.make_async_copy(typ_hbm.at[pl.ds(off, CH)], typs[b],
                              semL[b]).wait()
        pltpu.make_async_copy(inve_hbm.at[pl.ds(off, CH)], invs[b],
                              semL[b]).wait()

        def idx_body(j, c2):
            s = srcs[b][pl.ds(j * 16, 16)]
            t = typs[b][pl.ds(j * 16, 16)]
            idxs[b][pl.ds(j * 16, 16)] = s * R + t
            return c2
        lax.fori_loop(0, CH // 16, idx_body, 0)

    def scale_rows(b):
        def scale_body(g, c2):
            rid = g * 2 + rowq
            iv = plsc.load_gather(invs[b], [rid])
            val = plsc.load_gather(rows[b], [rid, colq])
            plsc.store_scatter(rows[b], [rid, colq], val * iv)
            return c2
        lax.fori_loop(0, CH // 2, scale_body, 0)

    def sub(b, k):
        prep_idx(1 - b, k + 1)
        pltpu.make_async_copy(xw_hbm.at[idxs[b]], rows[b], semG[b]).wait()
        scale_rows(b)
        pltpu.async_copy(xw_hbm.at[idxs[1 - b]], rows[1 - b], semG[1 - b])
        pltpu.sync_copy(rows[b], agg_sh.at[dsts[b]], add=True)

        @pl.when(k + 2 < _NCH2)
        def _():
            issue_linear(b, k + 2)

    issue_linear(0, 0)
    prep_idx(0, 0)
    pltpu.async_copy(xw_hbm.at[idxs[0]], rows[0], semG[0])
    issue_linear(1, 1)

    def pair_body(p, carry):
        sub(0, p * 2)
        sub(1, p * 2 + 1)
        return carry
    lax.fori_loop(0, (_NCH2 - 1) // 2, pair_body, 0)
    pltpu.make_async_copy(xw_hbm.at[idxs[0]], rows[0], semG[0]).wait()
    scale_rows(0)
    pltpu.sync_copy(rows[0], agg_sh.at[dsts[0]], add=True)
    plsc.subcore_barrier()

    @pl.when(sid < NS - 1)
    def _():
        pltpu.sync_copy(agg_sh.at[pl.ds(st, NROW)], obuf)
        pltpu.sync_copy(obuf, agg_out.at[cid, pl.ds(st, NROW)])

    @pl.when(sid == NS - 1)
    def _():
        pltpu.sync_copy(agg_sh.at[pl.ds(st, NROW_L)],
                        obuf.at[pl.ds(0, NROW_L)])
        pltpu.sync_copy(obuf.at[pl.ds(0, NROW_L)],
                        agg_out.at[cid, pl.ds(st, NROW_L)])


def _final_body(aggp_ref, h_ref, root2_ref, bias_ref, out_ref):
    a = (aggp_ref[0] + aggp_ref[1] + bias_ref[...]
         + jnp.dot(h_ref[...], root2_ref[...],
                   preferred_element_type=jnp.float32))
    m = jnp.max(a, axis=1, keepdims=True)
    ex = jnp.exp(a - m)
    lse = jnp.log(jnp.sum(ex, axis=1, keepdims=True))
    out_ref[...] = a - m - lse


def _final_call(agg2_p, h, root2, bias2_2d):
    blk = 2000
    return pl.pallas_call(
        _final_body,
        grid=(N // blk,),
        in_specs=[pl.BlockSpec((NC, blk, C), lambda i: (0, i, 0)),
                  pl.BlockSpec((blk, H), lambda i: (i, 0)),
                  pl.BlockSpec((H, C), lambda i: (0, 0)),
                  pl.BlockSpec((1, C), lambda i: (0, 0))],
        out_specs=pl.BlockSpec((blk, C), lambda i: (i, 0)),
        out_shape=jax.ShapeDtypeStruct((N, C), jnp.float32),
    )(agg2_p, h, root2, bias2_2d)


@jax.jit
def kernel(edge_index, edge_type, basis1, comp1, root1, bias1,
           basis2, comp2, root2, bias2):
    src = edge_index[0].astype(jnp.int32)
    dst = edge_index[1].astype(jnp.int32)
    typ = edge_type.astype(jnp.int32)

    b3 = basis1.transpose(0, 2, 1).reshape(NB * H, N)
    m2 = jnp.kron(comp1, jnp.eye(H, dtype=jnp.float32))
    w1_rows = _w1_call(b3, m2).reshape(N * R, H)
    cnt_p = _count_kernel(dst, typ)
    invn = _invn_call(cnt_p.reshape(NC, N, RH))
    agg1_p, inve = _agg1_kernel(src, dst, typ, w1_rows, invn)

    basis2_t = basis2.transpose(1, 0, 2).reshape(H, NB * C)
    m_mat = jnp.kron(comp2.T, jnp.eye(C, dtype=jnp.float32))
    h, xw = _mid_call(agg1_p, root1, bias1.reshape(1, H), basis2_t, m_mat)
    xw_rows = xw.reshape(N * R, C)

    agg2_p = _agg2_kernel(src, dst, typ, xw_rows, inve)
    return _final_call(agg2_p, h, root2, bias2.reshape(1, C))


# speedup vs baseline: 1.0946x; 1.0946x over previous
import functools

import jax
import jax.numpy as jnp
from jax import lax
from jax.experimental import pallas as pl
from jax.experimental.pallas import tpu as pltpu
from jax.experimental.pallas import tpu_sc as plsc

N = 50000
E = 1600000
R = 16
NB = 30
H = 16
C = 8

NC = 2
NS = 16
NW = NC * NS
RH = R // NC
EPW = E // NW
EPT = E // NS
CH = 400
CHC = 2000
NROW = 3128
NROW_L = N - (NS - 1) * NROW

_MESH = plsc.VectorSubcoreMesh(core_axis_name="c", subcore_axis_name="s",
                               num_cores=NC, num_subcores=NS)
_SC_PARAMS = pltpu.CompilerParams(needs_layout_passes=False,
                                  use_tc_tiling_on_sc=False)


def _w1_body(b3_ref, m2_ref, out_ref):
    out_ref[...] = lax.dot_general(
        b3_ref[...], m2_ref[...],
        ((((0,), (1,)), ((), ()))),
        preferred_element_type=jnp.float32)


def _w1_call(b3, m2):
    blk = 1280
    grid = (N + blk - 1) // blk
    return pl.pallas_call(
        _w1_body,
        grid=(grid,),
        in_specs=[pl.BlockSpec((NB * H, blk), lambda i: (0, i)),
                  pl.BlockSpec((R * H, NB * H), lambda i: (0, 0))],
        out_specs=pl.BlockSpec((blk, R * H), lambda i: (i, 0)),
        out_shape=jax.ShapeDtypeStruct((N, R * H), jnp.float32),
    )(b3, m2)


@functools.partial(
    pl.kernel,
    out_type=jax.ShapeDtypeStruct((NC * N * RH,), jnp.float32),
    mesh=_MESH,
    compiler_params=_SC_PARAMS,
    scratch_types=[
        pltpu.VMEM((CHC,), jnp.int32),
        pltpu.VMEM((CHC,), jnp.int32),
        pltpu.VMEM((CHC,), jnp.int32),
        pltpu.VMEM((CHC,), jnp.float32),
        pltpu.SemaphoreType.DMA,
        pltpu.SemaphoreType.DMA,
        pltpu.VMEM((25600,), jnp.float32),
        pltpu.VMEM_SHARED((N * RH,), jnp.float32),
    ],
)
def _count_kernel(dst_hbm, typ_hbm, out_hbm,
                  dst_v, typ_v, idx_v, ones_v, semA, semB, cbuf, cnt_sh):
    cid = lax.axis_index("c")
    sid = lax.axis_index("s")
    base = sid * EPT
    zsl = (N * RH) // NS

    def zero_body(j, carry):
        cbuf[pl.ds(j * 16, 16)] = jnp.zeros((16,), jnp.float32)
        return carry
    lax.fori_loop(0, 25600 // 16, zero_body, 0)

    def fill_body(j, carry):
        ones_v[pl.ds(j * 16, 16)] = jnp.full((16,), 1.0, jnp.float32)
        return carry
    lax.fori_loop(0, CHC // 16, fill_body, 0)
    pltpu.sync_copy(cbuf.at[pl.ds(0, zsl)], cnt_sh.at[pl.ds(sid * zsl, zsl)])
    plsc.subcore_barrier()

    tlo = cid * RH

    def chunk_body(k, carry):
        off = base + k * CHC
        cpa = pltpu.async_copy(dst_hbm.at[pl.ds(off, CHC)], dst_v, semA)
        cpb = pltpu.async_copy(typ_hbm.at[pl.ds(off, CHC)], typ_v, semB)
        cpa.wait()
        cpb.wait()

        def vec_body(j, c2):
            d = dst_v[pl.ds(j * 16, 16)]
            t = typ_v[pl.ds(j * 16, 16)] - tlo
            ok = (t >= 0) & (t < RH)
            idx_v[pl.ds(j * 16, 16)] = jnp.where(ok, d * RH + t, -1)
            return c2
        lax.fori_loop(0, CHC // 16, vec_body, 0)
        pltpu.sync_copy(
            ones_v,
            cnt_sh.at[plsc.Indices(idx_v, ignored_value=-1)],
            add=True)
        return carry
    lax.fori_loop(0, EPT // CHC, chunk_body, 0)
    plsc.subcore_barrier()
    pltpu.sync_copy(cnt_sh.at[pl.ds(sid * zsl, zsl)], cbuf.at[pl.ds(0, zsl)])
    pltpu.sync_copy(cbuf.at[pl.ds(0, zsl)],
                    out_hbm.at[pl.ds(cid * N * RH + sid * zsl, zsl)])


def _invn_body(cnt_ref, out_ref):
    lo = 1.0 / jnp.maximum(cnt_ref[0], 1.0)
    hi = 1.0 / jnp.maximum(cnt_ref[1], 1.0)
    out_ref[...] = jnp.concatenate([lo, hi], axis=1)


def _invn_call(cnt3):
    blk = 5000
    return pl.pallas_call(
        _invn_body,
        grid=(N // blk,),
        in_specs=[pl.BlockSpec((NC, blk, RH), lambda i: (0, i, 0))],
        out_specs=pl.BlockSpec((blk, R), lambda i: (i, 0)),
        out_shape=jax.ShapeDtypeStruct((N, R), jnp.float32),
    )(cnt3)


_NCH1 = EPW // CH


@functools.partial(
    pl.kernel,
    out_type=(jax.ShapeDtypeStruct((NC, N, H), jnp.float32),
              jax.ShapeDtypeStruct((E,), jnp.float32)),
    mesh=_MESH,
    compiler_params=_SC_PARAMS,
    scratch_types=(
        [pltpu.VMEM((CH,), jnp.int32)] * 8 +
        [pltpu.VMEM((CH, H), jnp.float32)] * 2 +
        [pltpu.VMEM((CH, R), jnp.float32)] * 2 +
        [pltpu.VMEM((CH,), jnp.float32)] * 2 +
        [pltpu.VMEM((NROW, H), jnp.float32),
         pltpu.VMEM_SHARED((N, H), jnp.float32)] +
        [pltpu.SemaphoreType.DMA] * 6
    ),
)
def _agg1_kernel(src_hbm, dst_hbm, typ_hbm, w1_hbm, invn_hbm,
                 agg_out, inve_out,
                 src0, src1, dst0, dst1, typ0, typ1, idx0, idx1,
                 rows0, rows1, invr0, invr1, inve0, inve1,
                 obuf, agg_sh, semL0, semL1, semI0, semI1, semW0, semW1):
    cid = lax.axis_index("c")
    sid = lax.axis_index("s")
    wid = sid * NC + cid
    base = wid * EPW
    st = sid * NROW
    srcs = [src0, src1]
    dsts = [dst0, dst1]
    typs = [typ0, typ1]
    idxs = [idx0, idx1]
    rows = [rows0, rows1]
    invr = [invr0, invr1]
    inve = [inve0, inve1]
    semL = [semL0, semL1]
    semI = [semI0, semI1]
    semW = [semW0, semW1]

    def zrow_body(r, carry):
        obuf[r, :] = jnp.zeros((16,), jnp.float32)
        return carry
    lax.fori_loop(0, NROW, zrow_body, 0)

    @pl.when(sid < NS - 1)
    def _():
        pltpu.sync_copy(obuf, agg_sh.at[pl.ds(st, NROW)])

    @pl.when(sid == NS - 1)
    def _():
        pltpu.sync_copy(obuf.at[pl.ds(0, NROW_L)],
                        agg_sh.at[pl.ds(st, NROW_L)])
    plsc.subcore_barrier()

    def issue_linear(b, k):
        off = base + k * CH
        pltpu.async_copy(src_hbm.at[pl.ds(off, CH)], srcs[b], semL[b])
        pltpu.async_copy(dst_hbm.at[pl.ds(off, CH)], dsts[b], semL[b])
        pltpu.async_copy(typ_hbm.at[pl.ds(off, CH)], typs[b], semL[b])

    def prep_idx(b, k):
        off = base + k * CH
        pltpu.make_async_copy(src_hbm.at[pl.ds(off, CH)], srcs[b],
                              semL[b]).wait()
        pltpu.make_async_copy(dst_hbm.at[pl.ds(off, CH)], dsts[b],
                              semL[b]).wait()
        pltpu.make_async_copy(typ_hbm.at[pl.ds(off, CH)], typs[b],
                              semL[b]).wait()

        def idx_body(jj, c2):
            s = srcs[b][pl.ds(jj * 16, 16)]
            t = typs[b][pl.ds(jj * 16, 16)]
            idxs[b][pl.ds(jj * 16, 16)] = s * R + t
            return c2
        lax.fori_loop(0, CH // 16, idx_body, 0)

    def issue_gathers(b):
        pltpu.async_copy(invn_hbm.at[dsts[b]], invr[b], semI[b])
        pltpu.async_copy(w1_hbm.at[idxs[b]], rows[b], semW[b])

    def finish(b, k):
        off = base + k * CH
        pltpu.make_async_copy(invn_hbm.at[dsts[b]], invr[b], semI[b]).wait()

        def ext_body(jj, c2):
            t = typs[b][pl.ds(jj * 16, 16)]
            r = jnp.arange(16, dtype=jnp.int32) + jj * 16
            inve[b][pl.ds(jj * 16, 16)] = plsc.load_gather(invr[b], [r, t])
            return c2
        lax.fori_loop(0, CH // 16, ext_body, 0)
        pltpu.make_async_copy(w1_hbm.at[idxs[b]], rows[b], semW[b]).wait()

        def scale_body(g, c2):
            iv = inve[b][pl.ds(g * 16, 16)]
            for l in range(16):
                e = g * 16 + l
                rows[b][e, :] = rows[b][e, :] * iv[l]
            return c2
        lax.fori_loop(0, CH // 16, scale_body, 0)
        return off

    def sub(b, k):
        prep_idx(1 - b, k + 1)
        off = finish(b, k)
        issue_gathers(1 - b)
        pltpu.sync_copy(rows[b], agg_sh.at[dsts[b]], add=True)
        pltpu.sync_copy(inve[b], inve_out.at[pl.ds(off, CH)])

        @pl.when(k + 2 < _NCH1)
        def _():
            issue_linear(b, k + 2)

    issue_linear(0, 0)
    prep_idx(0, 0)
    issue_gathers(0)
    issue_linear(1, 1)

    def pair_body(p, carry):
        sub(0, p * 2)
        sub(1, p * 2 + 1)
        return carry
    lax.fori_loop(0, (_NCH1 - 1) // 2, pair_body, 0)
    offl = finish(0, _NCH1 - 1)
    pltpu.sync_copy(rows[0], agg_sh.at[dsts[0]], add=True)
    pltpu.sync_copy(inve[0], inve_out.at[pl.ds(offl, CH)])
    plsc.subcore_barrier()

    @pl.when(sid < NS - 1)
    def _():
        pltpu.sync_copy(agg_sh.at[pl.ds(st, NROW)], obuf)
        pltpu.sync_copy(obuf, agg_out.at[cid, pl.ds(st, NROW)])

    @pl.when(sid == NS - 1)
    def _():
        pltpu.sync_copy(agg_sh.at[pl.ds(st, NROW_L)],
                        obuf.at[pl.ds(0, NROW_L)])
        pltpu.sync_copy(obuf.at[pl.ds(0, NROW_L)],
                        agg_out.at[cid, pl.ds(st, NROW_L)])


def _mid_body(aggp_ref, root_ref, bias_ref, basis2t_ref, m_ref,
              h_ref, xw_ref):
    h = jnp.maximum(
        aggp_ref[0] + aggp_ref[1] + root_ref[...] + bias_ref[...], 0.0)
    h_ref[...] = h
    hb = jnp.dot(h, basis2t_ref[...], preferred_element_type=jnp.float32)
    xw_ref[...] = jnp.dot(hb, m_ref[...], preferred_element_type=jnp.float32)


def _mid_call(agg1_p, root1, bias1_2d, basis2_t, m_mat):
    blk = 2000
    return pl.pallas_call(
        _mid_body,
        grid=(N // blk,),
        in_specs=[pl.BlockSpec((NC, blk, H), lambda i: (0, i, 0)),
                  pl.BlockSpec((blk, H), lambda i: (i, 0)),
                  pl.BlockSpec((1, H), lambda i: (0, 0)),
                  pl.BlockSpec((H, NB * C), lambda i: (0, 0)),
                  pl.BlockSpec((NB * C, R * C), lambda i: (0, 0))],
        out_specs=(pl.BlockSpec((blk, H), lambda i: (i, 0)),
                   pl.BlockSpec((blk, R * C), lambda i: (i, 0))),
        out_shape=(jax.ShapeDtypeStruct((N, H), jnp.float32),
                   jax.ShapeDtypeStruct((N, R * C), jnp.float32)),
    )(agg1_p, root1, bias1_2d, basis2_t, m_mat)


_NCH2 = EPW // CH


@functools.partial(
    pl.kernel,
    out_type=jax.ShapeDtypeStruct((NC, N, C), jnp.float32),
    mesh=_MESH,
    compiler_params=_SC_PARAMS,
    scratch_types=(
        [pltpu.VMEM((CH,), jnp.int32)] * 8 +
        [pltpu.VMEM((CH,), jnp.float32)] * 2 +
        [pltpu.VMEM((CH, C), jnp.float32)] * 2 +
        [pltpu.VMEM((NROW, C), jnp.float32),
         pltpu.VMEM_SHARED((N, C), jnp.float32)] +
        [pltpu.SemaphoreType.DMA] * 4
    ),
)
def _agg2_kernel(src_hbm, dst_hbm, typ_hbm, xw_hbm, inve_hbm,
                 agg_out,
                 src0, src1, dst0, dst1, typ0, typ1, idx0, idx1,
                 inv0, inv1, rows0, rows1, obuf, agg_sh,
                 semL0, semL1, semG0, semG1):
    cid = lax.axis_index("c")
    sid = lax.axis_index("s")
    wid = sid * NC + cid
    base = wid * EPW
    st = sid * NROW
    lane = jnp.arange(16, dtype=jnp.int32)
    colq = lane & 7
    rowq = lane >> 3
    srcs = [src0, src1]
    dsts = [dst0, dst1]
    typs = [typ0, typ1]
    idxs = [idx0, idx1]
    invs = [inv0, inv1]
    rows = [rows0, rows1]
    semL = [semL0, semL1]
    semG = [semG0, semG1]

    def zrow_body(g, carry):
        rid = g * 2 + rowq
        plsc.store_scatter(obuf, [rid, colq], jnp.zeros((16,), jnp.float32))
        return carry
    lax.fori_loop(0, NROW // 2, zrow_body, 0)

    @pl.when(sid < NS - 1)
    def _():
        pltpu.sync_copy(obuf, agg_sh.at[pl.ds(st, NROW)])

    @pl.when(sid == NS - 1)
    def _():
        pltpu.sync_copy(obuf.at[pl.ds(0, NROW_L)],
                        agg_sh.at[pl.ds(st, NROW_L)])
    plsc.subcore_barrier()

    def issue_linear(b, k):
        off = base + k * CH
        pltpu.async_copy(src_hbm.at[pl.ds(off, CH)], srcs[b], semL[b])
        pltpu.async_copy(dst_hbm.at[pl.ds(off, CH)], dsts[b], semL[b])
        pltpu.async_copy(typ_hbm.at[pl.ds(off, CH)], typs[b], semL[b])
        pltpu.async_copy(inve_hbm.at[pl.ds(off, CH)], invs[b], semL[b])

    def prep_idx(b, k):
        off = base + k * CH
        pltpu.make_async_copy(src_hbm.at[pl.ds(off, CH)], srcs[b],
                              semL[b]).wait()
        pltpu.make_async_copy(dst_hbm.at[pl.ds(off, CH)], dsts[b],
                              semL[b]).wait()
        pltpu.make_async_copy(typ_hbm.at[pl.ds(off, CH)], typs[b],
                              semL[b]).wait()
        pltpu.make_async_copy(inve_hbm.at[pl.ds(off, CH)], invs[b],
                              semL[b]).wait()

        def idx_body(j, c2):
            s = srcs[b][pl.ds(j * 16, 16)]
            t = typs[b][pl.ds(j * 16, 16)]
            idxs[b][pl.ds(j * 16, 16)] = s * R + t
            return c2
        lax.fori_loop(0, CH // 16, idx_body, 0)

    def scale_rows(b):
        def scale_body(g, c2):
            rid = g * 2 + rowq
            iv = plsc.load_gather(invs[b], [rid])
            val = plsc.load_gather(rows[b], [rid, colq])
            plsc.store_scatter(rows[b], [rid, colq], val * iv)
            return c2
        lax.fori_loop(0, CH // 2, scale_body, 0)

    def sub(b, k):
        prep_idx(1 - b, k + 1)
        pltpu.make_async_copy(xw_hbm.at[idxs[b]], rows[b], semG[b]).wait()
        scale_rows(b)
        pltpu.async_copy(xw_hbm.at[idxs[1 - b]], rows[1 - b], semG[1 - b])
        pltpu.sync_copy(rows[b], agg_sh.at[dsts[b]], add=True)

        @pl.when(k + 2 < _NCH2)
        def _():
            issue_linear(b, k + 2)

    issue_linear(0, 0)
    prep_idx(0, 0)
    pltpu.async_copy(xw_hbm.at[idxs[0]], rows[0], semG[0])
    issue_linear(1, 1)

    def pair_body(p, carry):
        sub(0, p * 2)
        sub(1, p * 2 + 1)
        return carry
    lax.fori_loop(0, (_NCH2 - 1) // 2, pair_body, 0)
    pltpu.make_async_copy(xw_hbm.at[idxs[0]], rows[0], semG[0]).wait()
    scale_rows(0)
    pltpu.sync_copy(rows[0], agg_sh.at[dsts[0]], add=True)
    plsc.subcore_barrier()

    @pl.when(sid < NS - 1)
    def _():
        pltpu.sync_copy(agg_sh.at[pl.ds(st, NROW)], obuf)
        pltpu.sync_copy(obuf, agg_out.at[cid, pl.ds(st, NROW)])

    @pl.when(sid == NS - 1)
    def _():
        pltpu.sync_copy(agg_sh.at[pl.ds(st, NROW_L)],
                        obuf.at[pl.ds(0, NROW_L)])
        pltpu.sync_copy(obuf.at[pl.ds(0, NROW_L)],
                        agg_out.at[cid, pl.ds(st, NROW_L)])


def _final_body(aggp_ref, h_ref, root2_ref, bias_ref, out_ref):
    a = (aggp_ref[0] + aggp_ref[1] + bias_ref[...]
         + jnp.dot(h_ref[...], root2_ref[...],
                   preferred_element_type=jnp.float32))
    m = jnp.max(a, axis=1, keepdims=True)
    ex = jnp.exp(a - m)
    lse = jnp.log(jnp.sum(ex, axis=1, keepdims=True))
    out_ref[...] = a - m - lse


def _final_call(agg2_p, h, root2, bias2_2d):
    blk = 2000
    return pl.pallas_call(
        _final_body,
        grid=(N // blk,),
        in_specs=[pl.BlockSpec((NC, blk, C), lambda i: (0, i, 0)),
                  pl.BlockSpec((blk, H), lambda i: (i, 0)),
                  pl.BlockSpec((H, C), lambda i: (0, 0)),
                  pl.BlockSpec((1, C), lambda i: (0, 0))],
        out_specs=pl.BlockSpec((blk, C), lambda i: (i, 0)),
        out_shape=jax.ShapeDtypeStruct((N, C), jnp.float32),
    )(agg2_p, h, root2, bias2_2d)


@jax.jit
def kernel(edge_index, edge_type, basis1, comp1, root1, bias1,
           basis2, comp2, root2, bias2):
    src = edge_index[0].astype(jnp.int32)
    dst = edge_index[1].astype(jnp.int32)
    typ = edge_type.astype(jnp.int32)

    b3 = basis1.transpose(0, 2, 1).reshape(NB * H, N)
    m2 = jnp.kron(comp1, jnp.eye(H, dtype=jnp.float32))
    w1_rows = _w1_call(b3, m2).reshape(N * R, H)
    cnt_p = _count_kernel(dst, typ)
    invn = _invn_call(cnt_p.reshape(NC, N, RH))
    agg1_p, inve = _agg1_kernel(src, dst, typ, w1_rows, invn)

    basis2_t = basis2.transpose(1, 0, 2).reshape(H, NB * C)
    m_mat = jnp.kron(comp2.T, jnp.eye(C, dtype=jnp.float32))
    h, xw = _mid_call(agg1_p, root1, bias1.reshape(1, H), basis2_t, m_mat)
    xw_rows = xw.reshape(N * R, C)

    agg2_p = _agg2_kernel(src, dst, typ, xw_rows, inve)
    return _final_call(agg2_p, h, root2, bias2.reshape(1, C))
